# baseline (speedup 1.0000x reference)
import functools

import jax
import jax.numpy as jnp
from jax.experimental import pallas as pl
from jax.experimental.pallas import tpu as pltpu

N = 16384
BI = 256
BJ = 2048


def _body(s_ref, t_ref, e_ref, out_ref, w_ref, aux_ref):
    k = pl.program_id(0)
    nblocks = pl.num_programs(0)

    @pl.when(k == 0)
    def _init():
        smax = jnp.max(s_ref[...])
        aux_ref[0, 0] = smax
        aux_ref[0, 1] = 0.0
        w_ref[...] = jnp.exp(s_ref[...] - smax)

    smax = aux_ref[0, 0]

    t_i = t_ref[0, pl.ds(k * BI, BI)].reshape(BI, 1)
    i_idx = (k * BI + jax.lax.broadcasted_iota(jnp.int32, (BI, 1), 0))

    r_acc = jnp.zeros((BI, 1), dtype=jnp.float32)
    for c in range(N // BJ):
        t_j = t_ref[0, pl.ds(c * BJ, BJ)].reshape(1, BJ)
        w_j = w_ref[0, pl.ds(c * BJ, BJ)].reshape(1, BJ)
        j_idx = c * BJ + jax.lax.broadcasted_iota(jnp.int32, (1, BJ), 1)
        in_risk = (t_j > t_i) | ((t_j == t_i) & (j_idx <= i_idx))
        r_acc = r_acc + jnp.sum(
            jnp.where(in_risk, w_j, 0.0), axis=1, keepdims=True)

    s_i = s_ref[0, pl.ds(k * BI, BI)].reshape(BI, 1)
    e_i = e_ref[0, pl.ds(k * BI, BI)].reshape(BI, 1)
    contrib = jnp.sum(e_i * (s_i - smax - jnp.log(r_acc)))
    total = aux_ref[0, 1] + contrib
    aux_ref[0, 1] = total

    @pl.when(k == nblocks - 1)
    def _fin():
        out_ref[...] = jnp.broadcast_to(-total / N, (1, 1))


@jax.jit
def _cox_loss(s_row, t_row, e_row):
    out = pl.pallas_call(
        _body,
        grid=(N // BI,),
        in_specs=[
            pl.BlockSpec((1, N), lambda k: (0, 0)),
            pl.BlockSpec((1, N), lambda k: (0, 0)),
            pl.BlockSpec((1, N), lambda k: (0, 0)),
        ],
        out_specs=pl.BlockSpec((1, 1), lambda k: (0, 0)),
        out_shape=jax.ShapeDtypeStruct((1, 1), jnp.float32),
        scratch_shapes=[
            pltpu.VMEM((1, N), jnp.float32),
            pltpu.SMEM((1, 2), jnp.float32),
        ],
    )(s_row, t_row, e_row)
    return out[0, 0]


def kernel(scores, truth):
    s_row = scores.reshape(1, N)
    e_row = truth[:, 0].reshape(1, N)
    t_row = truth[:, 1].reshape(1, N)
    return _cox_loss(s_row, t_row, e_row)
